# Initial kernel scaffold; baseline (speedup 1.0000x reference)
import functools

import jax
import jax.numpy as jnp
from jax import lax
from jax.experimental import pallas as pl
from jax.experimental.pallas import tpu as pltpu
from jax.experimental.pallas import tpu_sc as plsc

N = 10000
E = 320000
N_PAD = 10240
C = 128
NW = 32
CH = 80
E_PAD = NW * C * CH
RB = 256
NBLK = N_PAD // RB
ROWS_PER_TILE = N_PAD // 16

_f32 = jnp.float32


def _mesh():
    return plsc.VectorSubcoreMesh(core_axis_name="c", subcore_axis_name="s")


def _fill_const(ref, rows, d, val):
    nv = d // 16

    def row(i, _):
        for j in range(nv):
            ref[i, pl.ds(j * 16, 16)] = jnp.full((16,), val, _f32)
        return 0

    lax.fori_loop(0, rows, row, 0)


def _sc_degree(src3):
    @functools.partial(
        pl.kernel,
        mesh=_mesh(),
        out_type=jax.ShapeDtypeStruct((2, N_PAD, 16), _f32),
        scratch_types=[
            pltpu.VMEM((CH, C), jnp.int32),
            pltpu.VMEM((C, 16), _f32),
            pltpu.VMEM((C, 16), _f32),
            pltpu.VMEM_SHARED((N_PAD, 16), _f32),
        ],
    )
    def k(src_hbm, out_hbm, idx_v, ones_v, zer_v, acc):
        cid = lax.axis_index("c")
        sid = lax.axis_index("s")
        wid = sid * 2 + cid
        _fill_const(ones_v, C, 16, 1.0)
        _fill_const(zer_v, C, 16, 0.0)
        base = sid * ROWS_PER_TILE

        def zrow(j, _):
            pltpu.sync_copy(zer_v, acc.at[pl.ds(base + j * C, C)])
            return 0

        lax.fori_loop(0, ROWS_PER_TILE // C, zrow, 0)
        plsc.subcore_barrier()

        pltpu.sync_copy(src_hbm.at[wid], idx_v)

        def body(g, _):
            pltpu.sync_copy(ones_v, acc.at[idx_v.at[g]], add=True)
            return 0

        lax.fori_loop(0, CH, body, 0)
        plsc.subcore_barrier()
        pltpu.sync_copy(acc.at[pl.ds(base, ROWS_PER_TILE)],
                        out_hbm.at[cid, pl.ds(base, ROWS_PER_TILE)])

    return k(src3)


def _sc_prop(u, src3, dst3, d):
    @functools.partial(
        pl.kernel,
        mesh=_mesh(),
        out_type=jax.ShapeDtypeStruct((2, N_PAD, d), _f32),
        scratch_types=[
            pltpu.VMEM((CH, C), jnp.int32),
            pltpu.VMEM((CH, C), jnp.int32),
            pltpu.VMEM((C, d), _f32),
            pltpu.VMEM((C, d), _f32),
            pltpu.VMEM((C, d), _f32),
            pltpu.VMEM_SHARED((N_PAD, d), _f32),
            pltpu.SemaphoreType.DMA,
            pltpu.SemaphoreType.DMA,
        ],
    )
    def k(u_hbm, src_hbm, dst_hbm, out_hbm,
          src_v, dst_v, g0, g1, zer_v, acc, sem0, sem1):
        cid = lax.axis_index("c")
        sid = lax.axis_index("s")
        wid = sid * 2 + cid
        _fill_const(zer_v, C, d, 0.0)
        base = sid * ROWS_PER_TILE

        def zrow(j, _):
            pltpu.sync_copy(zer_v, acc.at[pl.ds(base + j * C, C)])
            return 0

        lax.fori_loop(0, ROWS_PER_TILE // C, zrow, 0)
        plsc.subcore_barrier()

        pltpu.sync_copy(src_hbm.at[wid], src_v)
        pltpu.sync_copy(dst_hbm.at[wid], dst_v)

        def body(g, _):
            e = g * 2
            cpa = pltpu.async_copy(u_hbm.at[src_v.at[e]], g0, sem0)
            cpb = pltpu.async_copy(u_hbm.at[src_v.at[e + 1]], g1, sem1)
            cpa.wait()
            pltpu.sync_copy(g0, acc.at[dst_v.at[e]], add=True)
            cpb.wait()
            pltpu.sync_copy(g1, acc.at[dst_v.at[e + 1]], add=True)
            return 0

        lax.fori_loop(0, CH // 2, body, 0)
        plsc.subcore_barrier()
        pltpu.sync_copy(acc.at[pl.ds(base, ROWS_PER_TILE)],
                        out_hbm.at[cid, pl.ds(base, ROWS_PER_TILE)])

    return k(u, src3, dst3)


def _row_spec(d):
    return pl.BlockSpec((RB, d), lambda i: (i, 0))


def _full_spec(shape):
    nd = len(shape)
    return pl.BlockSpec(shape, lambda i, _n=nd: (0,) * _n)


def _tc_prep(xp, pp, hist):
    def body(x_ref, p_ref, h_ref, u_ref, dis2_ref, sdeg_ref):
        deg = h_ref[0, :, 0:1] + h_ref[1, :, 0:1] + 1.0
        dis = lax.rsqrt(deg)
        u_ref[:, 0:128] = x_ref[...] * dis
        u_ref[:, 128:144] = p_ref[...] * dis
        dis2_ref[...] = 1.0 / deg
        sdeg_ref[...] = jnp.sqrt(deg)

    return pl.pallas_call(
        body,
        grid=(NBLK,),
        in_specs=[
            _row_spec(128),
            _row_spec(16),
            pl.BlockSpec((2, RB, 16), lambda i: (0, i, 0)),
        ],
        out_specs=[_row_spec(144), _row_spec(1), _row_spec(1)],
        out_shape=[
            jax.ShapeDtypeStruct((N_PAD, 144), _f32),
            jax.ShapeDtypeStruct((N_PAD, 1), _f32),
            jax.ShapeDtypeStruct((N_PAD, 1), _f32),
        ],
    )(xp, pp, hist)


def _tc_combine(p3, ukm1, ukm2, dis2, alpha, d):
    has_prev = ukm2 is not None

    def body(*refs):
        if has_prev:
            p_ref, a_ref, b_ref, s_ref, o_ref = refs
        else:
            p_ref, a_ref, s_ref, o_ref = refs
        y = p_ref[0] + p_ref[1] + a_ref[...]
        o = (-alpha) * s_ref[...] * y
        if has_prev:
            o = o - b_ref[...]
        o_ref[...] = o

    in_specs = [pl.BlockSpec((2, RB, d), lambda i: (0, i, 0)), _row_spec(d)]
    args = [p3, ukm1]
    if has_prev:
        in_specs.append(_row_spec(d))
        args.append(ukm2)
    in_specs.append(_row_spec(1))
    args.append(dis2)
    return pl.pallas_call(
        body,
        grid=(NBLK,),
        in_specs=in_specs,
        out_specs=_row_spec(d),
        out_shape=jax.ShapeDtypeStruct((N_PAD, d), _f32),
    )(*args)


def _tc_layer1_final(us, sdeg, dis2, wc, b):
    def body(u0, u1, u2, u3, s_ref, s2_ref, w_ref, b_ref, o_ref):
        s = s_ref[...]
        acc = jnp.dot(u0[...] * s, w_ref[0:144, :],
                      preferred_element_type=_f32)
        acc += jnp.dot(u1[...] * s, w_ref[144:288, :],
                       preferred_element_type=_f32)
        acc += jnp.dot(u2[...] * s, w_ref[288:432, :],
                       preferred_element_type=_f32)
        acc += jnp.dot(u3[...] * s, w_ref[432:576, :],
                       preferred_element_type=_f32)
        h = jnp.maximum(acc + b_ref[...], 0.0)
        o_ref[...] = h * jnp.sqrt(s2_ref[...])

    return pl.pallas_call(
        body,
        grid=(NBLK,),
        in_specs=[_row_spec(144)] * 4
        + [_row_spec(1), _row_spec(1), _full_spec((576, 128)),
           _full_spec((1, 128))],
        out_specs=_row_spec(128),
        out_shape=jax.ShapeDtypeStruct((N_PAD, 128), _f32),
    )(*us, sdeg, dis2, wc, b)


def _tc_layer2_final(us, sdeg, wc, b, wmu, bmu, wlv, blv):
    def body(u0, u1, u2, u3, s_ref, w_ref, b_ref,
             wmu_ref, bmu_ref, wlv_ref, blv_ref, mu_ref, lv_ref):
        s = s_ref[...]
        acc = jnp.dot(u0[...] * s, w_ref[0:128, :],
                      preferred_element_type=_f32)
        acc += jnp.dot(u1[...] * s, w_ref[128:256, :],
                       preferred_element_type=_f32)
        acc += jnp.dot(u2[...] * s, w_ref[256:384, :],
                       preferred_element_type=_f32)
        acc += jnp.dot(u3[...] * s, w_ref[384:512, :],
                       preferred_element_type=_f32)
        h = jnp.maximum(acc + b_ref[...], 0.0)
        mu_ref[...] = jnp.dot(h, wmu_ref[...],
                              preferred_element_type=_f32) + bmu_ref[...]
        lv_ref[...] = jnp.dot(h, wlv_ref[...],
                              preferred_element_type=_f32) + blv_ref[...]

    return pl.pallas_call(
        body,
        grid=(NBLK,),
        in_specs=[_row_spec(128)] * 4
        + [_row_spec(1), _full_spec((512, 128)), _full_spec((1, 128)),
           _full_spec((128, 64)), _full_spec((1, 64)),
           _full_spec((128, 64)), _full_spec((1, 64))],
        out_specs=[_row_spec(64), _row_spec(64)],
        out_shape=[
            jax.ShapeDtypeStruct((N_PAD, 64), _f32),
            jax.ShapeDtypeStruct((N_PAD, 64), _f32),
        ],
    )(*us, sdeg, wc, b, wmu, bmu, wlv, blv)


def kernel(x, edge_index, lap_pe, W1, b1, W2, b2, Wmu, bmu, Wlv, blv):
    pad_idx = jnp.full((E_PAD - E,), N, jnp.int32)
    src3 = jnp.concatenate([edge_index[0], pad_idx]).reshape(NW, CH, C)
    dst3 = jnp.concatenate([edge_index[1], pad_idx]).reshape(NW, CH, C)
    xp = jnp.pad(x, ((0, N_PAD - N), (0, 0)))
    pp = jnp.pad(lap_pe, ((0, N_PAD - N), (0, 0)))

    hist = _sc_degree(src3)
    u0, dis2, sdeg = _tc_prep(xp, pp, hist)

    p = _sc_prop(u0, src3, dst3, 144)
    u1 = _tc_combine(p, u0, None, dis2, 1.0, 144)
    p = _sc_prop(u1, src3, dst3, 144)
    u2 = _tc_combine(p, u1, u0, dis2, 2.0, 144)
    p = _sc_prop(u2, src3, dst3, 144)
    u3 = _tc_combine(p, u2, u1, dis2, 2.0, 144)

    v0 = _tc_layer1_final((u0, u1, u2, u3), sdeg, dis2,
                          W1.reshape(4 * 144, 128), b1.reshape(1, 128))

    p = _sc_prop(v0, src3, dst3, 128)
    v1 = _tc_combine(p, v0, None, dis2, 1.0, 128)
    p = _sc_prop(v1, src3, dst3, 128)
    v2 = _tc_combine(p, v1, v0, dis2, 2.0, 128)
    p = _sc_prop(v2, src3, dst3, 128)
    v3 = _tc_combine(p, v2, v1, dis2, 2.0, 128)

    mu, lv = _tc_layer2_final((v0, v1, v2, v3), sdeg,
                              W2.reshape(4 * 128, 128), b2.reshape(1, 128),
                              Wmu, bmu.reshape(1, 64),
                              Wlv, blv.reshape(1, 64))
    return mu[:N], lv[:N]


# speedup vs baseline: 10.9558x; 10.9558x over previous
import functools

import jax
import jax.numpy as jnp
from jax import lax
from jax.experimental import pallas as pl
from jax.experimental.pallas import tpu as pltpu
from jax.experimental.pallas import tpu_sc as plsc

N = 10000
E = 320000
N_PAD = 10240
C = 128
CH = 157
E_PAD = 16 * CH * C
RB = 256
NBLK = N_PAD // RB
ROWS_PER_TILE = N_PAD // 16
W1H = 80
W2H = 64

_f32 = jnp.float32


def _mesh():
    return plsc.VectorSubcoreMesh(core_axis_name="c", subcore_axis_name="s")


def _sc_params():
    return pltpu.CompilerParams(use_tc_tiling_on_sc=False)


def _fill_const(ref, rows, d, val):
    nv = d // 16

    def row(i, _):
        for j in range(nv):
            ref[i, pl.ds(j * 16, 16)] = jnp.full((16,), val, _f32)
        return 0

    lax.fori_loop(0, rows, row, 0)


def _sc_degree(src3):
    @functools.partial(
        pl.kernel,
        mesh=_mesh(),
        compiler_params=_sc_params(),
        out_type=jax.ShapeDtypeStruct((2, N_PAD, 16), _f32),
        scratch_types=[
            pltpu.VMEM((CH, C), jnp.int32),
            pltpu.VMEM((C, 16), _f32),
            pltpu.VMEM((C, 16), _f32),
            pltpu.VMEM_SHARED((N_PAD, 16), _f32),
        ],
    )
    def k(src_hbm, out_hbm, idx_v, ones_v, zer_v, acc):
        cid = lax.axis_index("c")
        sid = lax.axis_index("s")
        _fill_const(ones_v, C, 16, 1.0)
        _fill_const(zer_v, C, 16, 0.0)
        base = sid * ROWS_PER_TILE

        def zrow(j, _):
            pltpu.sync_copy(zer_v, acc.at[pl.ds(base + j * C, C)])
            return 0

        lax.fori_loop(0, ROWS_PER_TILE // C, zrow, 0)
        plsc.subcore_barrier()

        pltpu.sync_copy(src_hbm.at[sid], idx_v)
        lo = cid * (CH // 2)
        hi = lo + (CH // 2) + cid * (CH % 2)

        def body(g, _):
            pltpu.sync_copy(ones_v, acc.at[idx_v.at[g]], add=True)
            return 0

        lax.fori_loop(lo, hi, body, 0)
        plsc.subcore_barrier()
        pltpu.sync_copy(acc.at[pl.ds(base, ROWS_PER_TILE)],
                        out_hbm.at[cid, pl.ds(base, ROWS_PER_TILE)])

    return k(src3)


def _sc_prop(u2, src3, dst3, w):
    @functools.partial(
        pl.kernel,
        mesh=_mesh(),
        compiler_params=_sc_params(),
        out_type=jax.ShapeDtypeStruct((2, N_PAD, w), _f32),
        scratch_types=[
            pltpu.VMEM((CH, C), jnp.int32),
            pltpu.VMEM((CH, C), jnp.int32),
            pltpu.VMEM((C, w), _f32),
            pltpu.VMEM((C, w), _f32),
            pltpu.VMEM_SHARED((N_PAD, w), _f32),
            pltpu.SemaphoreType.DMA,
            pltpu.SemaphoreType.DMA,
        ],
    )
    def k(u_hbm, src_hbm, dst_hbm, out_hbm,
          src_v, dst_v, g0, g1, acc, sem0, sem1):
        cid = lax.axis_index("c")
        sid = lax.axis_index("s")
        base = sid * ROWS_PER_TILE

        _fill_const(g0, C, w, 0.0)

        def zrow(j, _):
            pltpu.sync_copy(g0, acc.at[pl.ds(base + j * C, C)])
            return 0

        lax.fori_loop(0, ROWS_PER_TILE // C, zrow, 0)
        plsc.subcore_barrier()

        pltpu.sync_copy(src_hbm.at[sid], src_v)
        pltpu.sync_copy(dst_hbm.at[sid], dst_v)
        tab = u_hbm.at[cid]

        def body(g, _):
            e = g * 2
            cpa = pltpu.async_copy(tab.at[src_v.at[e]], g0, sem0)
            cpb = pltpu.async_copy(tab.at[src_v.at[e + 1]], g1, sem1)
            cpa.wait()
            pltpu.sync_copy(g0, acc.at[dst_v.at[e]], add=True)
            cpb.wait()
            pltpu.sync_copy(g1, acc.at[dst_v.at[e + 1]], add=True)
            return 0

        lax.fori_loop(0, CH // 2, body, 0)
        e = CH - 1
        cpa = pltpu.async_copy(tab.at[src_v.at[e]], g0, sem0)
        cpa.wait()
        pltpu.sync_copy(g0, acc.at[dst_v.at[e]], add=True)

        plsc.subcore_barrier()
        pltpu.sync_copy(acc.at[pl.ds(base, ROWS_PER_TILE)],
                        out_hbm.at[cid, pl.ds(base, ROWS_PER_TILE)])

    return k(u2, src3, dst3)


def _row_spec(d):
    return pl.BlockSpec((RB, d), lambda i: (i, 0))


def _split_spec(w):
    return pl.BlockSpec((2, RB, w), lambda i: (0, i, 0))


def _full_spec(shape):
    nd = len(shape)
    return pl.BlockSpec(shape, lambda i, _n=nd: (0,) * _n)


def _tc_prep(xp, pp, hist):
    def body(x_ref, p_ref, h_ref, u_ref, dis2_ref, sdeg_ref):
        deg = h_ref[0, :, 0:1] + h_ref[1, :, 0:1] + 1.0
        dis = lax.rsqrt(deg)
        u_ref[0, :, :] = x_ref[:, 0:80] * dis
        u_ref[1, :, 0:48] = x_ref[:, 80:128] * dis
        u_ref[1, :, 48:64] = p_ref[...] * dis
        u_ref[1, :, 64:80] = jnp.zeros((RB, 16), _f32)
        dis2_ref[...] = 1.0 / deg
        sdeg_ref[...] = jnp.sqrt(deg)

    return pl.pallas_call(
        body,
        grid=(NBLK,),
        in_specs=[
            _row_spec(128),
            _row_spec(16),
            pl.BlockSpec((2, RB, 16), lambda i: (0, i, 0)),
        ],
        out_specs=[_split_spec(W1H), _row_spec(1), _row_spec(1)],
        out_shape=[
            jax.ShapeDtypeStruct((2, N_PAD, W1H), _f32),
            jax.ShapeDtypeStruct((N_PAD, 1), _f32),
            jax.ShapeDtypeStruct((N_PAD, 1), _f32),
        ],
    )(xp, pp, hist)


def _tc_combine(y2, ukm1, ukm2, dis2, alpha, w):
    has_prev = ukm2 is not None

    def body(*refs):
        if has_prev:
            y_ref, a_ref, b_ref, s_ref, o_ref = refs
        else:
            y_ref, a_ref, s_ref, o_ref = refs
        s = s_ref[...]
        for c in (0, 1):
            o = (-alpha) * s * (y_ref[c] + a_ref[c])
            if has_prev:
                o = o - b_ref[c]
            o_ref[c, :, :] = o

    in_specs = [_split_spec(w), _split_spec(w)]
    args = [y2, ukm1]
    if has_prev:
        in_specs.append(_split_spec(w))
        args.append(ukm2)
    in_specs.append(_row_spec(1))
    args.append(dis2)
    return pl.pallas_call(
        body,
        grid=(NBLK,),
        in_specs=in_specs,
        out_specs=_split_spec(w),
        out_shape=jax.ShapeDtypeStruct((2, N_PAD, w), _f32),
    )(*args)


def _tc_layer1_final(us, sdeg, dis2, wc, b):
    def body(u0, u1, u2, u3, s_ref, s2_ref, w_ref, b_ref, o_ref):
        s = s_ref[...]
        acc = b_ref[...] + jnp.zeros((RB, 128), _f32)
        for k, u in enumerate((u0, u1, u2, u3)):
            acc += jnp.dot(u[0] * s, w_ref[pl.ds(144 * k, 80)],
                           preferred_element_type=_f32)
            acc += jnp.dot(u[1, :, 0:64] * s, w_ref[pl.ds(144 * k + 80, 64)],
                           preferred_element_type=_f32)
        h = jnp.maximum(acc, 0.0)
        dis = jnp.sqrt(s2_ref[...])
        o_ref[0, :, :] = h[:, 0:64] * dis
        o_ref[1, :, :] = h[:, 64:128] * dis

    return pl.pallas_call(
        body,
        grid=(NBLK,),
        in_specs=[_split_spec(W1H)] * 4
        + [_row_spec(1), _row_spec(1), _full_spec((576, 128)),
           _full_spec((1, 128))],
        out_specs=_split_spec(W2H),
        out_shape=jax.ShapeDtypeStruct((2, N_PAD, W2H), _f32),
    )(*us, sdeg, dis2, wc, b)


def _tc_layer2_final(us, sdeg, wc, b, wmu, bmu, wlv, blv):
    def body(u0, u1, u2, u3, s_ref, w_ref, b_ref,
             wmu_ref, bmu_ref, wlv_ref, blv_ref, mu_ref, lv_ref):
        s = s_ref[...]
        acc = b_ref[...] + jnp.zeros((RB, 128), _f32)
        for k, u in enumerate((u0, u1, u2, u3)):
            acc += jnp.dot(u[0] * s, w_ref[pl.ds(128 * k, 64)],
                           preferred_element_type=_f32)
            acc += jnp.dot(u[1] * s, w_ref[pl.ds(128 * k + 64, 64)],
                           preferred_element_type=_f32)
        h = jnp.maximum(acc, 0.0)
        mu_ref[...] = jnp.dot(h, wmu_ref[...],
                              preferred_element_type=_f32) + bmu_ref[...]
        lv_ref[...] = jnp.dot(h, wlv_ref[...],
                              preferred_element_type=_f32) + blv_ref[...]

    return pl.pallas_call(
        body,
        grid=(NBLK,),
        in_specs=[_split_spec(W2H)] * 4
        + [_row_spec(1), _full_spec((512, 128)), _full_spec((1, 128)),
           _full_spec((128, 64)), _full_spec((1, 64)),
           _full_spec((128, 64)), _full_spec((1, 64))],
        out_specs=[_row_spec(64), _row_spec(64)],
        out_shape=[
            jax.ShapeDtypeStruct((N_PAD, 64), _f32),
            jax.ShapeDtypeStruct((N_PAD, 64), _f32),
        ],
    )(*us, sdeg, wc, b, wmu, bmu, wlv, blv)


def kernel(x, edge_index, lap_pe, W1, b1, W2, b2, Wmu, bmu, Wlv, blv):
    pad_idx = jnp.full((E_PAD - E,), N, jnp.int32)
    src3 = jnp.concatenate([edge_index[0], pad_idx]).reshape(16, CH, C)
    dst3 = jnp.concatenate([edge_index[1], pad_idx]).reshape(16, CH, C)
    xp = jnp.pad(x, ((0, N_PAD - N), (0, 0)))
    pp = jnp.pad(lap_pe, ((0, N_PAD - N), (0, 0)))

    hist = _sc_degree(src3)
    u0, dis2, sdeg = _tc_prep(xp, pp, hist)

    y = _sc_prop(u0, src3, dst3, W1H)
    u1 = _tc_combine(y, u0, None, dis2, 1.0, W1H)
    y = _sc_prop(u1, src3, dst3, W1H)
    u2 = _tc_combine(y, u1, u0, dis2, 2.0, W1H)
    y = _sc_prop(u2, src3, dst3, W1H)
    u3 = _tc_combine(y, u2, u1, dis2, 2.0, W1H)

    v0 = _tc_layer1_final((u0, u1, u2, u3), sdeg, dis2,
                          W1.reshape(4 * 144, 128), b1.reshape(1, 128))

    y = _sc_prop(v0, src3, dst3, W2H)
    v1 = _tc_combine(y, v0, None, dis2, 1.0, W2H)
    y = _sc_prop(v1, src3, dst3, W2H)
    v2 = _tc_combine(y, v1, v0, dis2, 2.0, W2H)
    y = _sc_prop(v2, src3, dst3, W2H)
    v3 = _tc_combine(y, v2, v1, dis2, 2.0, W2H)

    mu, lv = _tc_layer2_final((v0, v1, v2, v3), sdeg,
                              W2.reshape(4 * 128, 128), b2.reshape(1, 128),
                              Wmu, bmu.reshape(1, 64),
                              Wlv, blv.reshape(1, 64))
    return mu[:N], lv[:N]
